# baseline (speedup 1.0000x reference)
import functools

import jax
import jax.numpy as jnp
from jax import lax
from jax.experimental import pallas as pl
from jax.experimental.pallas import tpu as pltpu
from jax.experimental.pallas import tpu_sc as plsc

N = 10000
NE = 320000
D = 128
H = 64
C = 80
CG = 40
EB = 1600
GRID = NE // EB
NW = 32
EPW = NE // NW
GCH = EPW // CG
EPT = NE // 16
SCH = EPT // C
NH = 5120
NHP = NH + 8
NP = 2 * NH
RPT = NH // 16
ZR = 8


def _node_body(h_ref, norm_ref, aw, ab, bw, bb, dw, db, ew, eb,
               ah_out, bh_out, dh_out, eh_out):
    hn = h_ref[...] * norm_ref[...]
    f32 = jnp.float32
    ah_out[...] = jnp.dot(hn, aw[...], preferred_element_type=f32) + ab[...]
    bh_out[...] = jnp.dot(hn, bw[...], preferred_element_type=f32) + bb[...]
    dh_out[...] = jnp.dot(hn, dw[...], preferred_element_type=f32) + db[...]
    eh_out[...] = jnp.dot(hn, ew[...], preferred_element_type=f32) + eb[...]


def _node_call(h, norm, aw, ab, bw, bb, dw, db, ew, eb):
    return pl.pallas_call(
        _node_body,
        out_shape=(
            jax.ShapeDtypeStruct((N, D), jnp.float32),
            jax.ShapeDtypeStruct((N, D), jnp.float32),
            jax.ShapeDtypeStruct((N, D), jnp.float32),
            jax.ShapeDtypeStruct((N, D), jnp.float32),
        ),
    )(h, norm, aw, ab, bw, bb, dw, db, ew, eb)


def _gather_body(dh, eh, src1, dst1, g_out,
                 srcc0, srcc1, dstc0, dstc1,
                 bufd0, bufd1, bufe0, bufe1,
                 semi0, semi1, semg0, semg1):
    cid = lax.axis_index("c")
    sid = lax.axis_index("s")
    wid = sid * 2 + cid
    ebase0 = wid * EPW
    srcc = (srcc0, srcc1)
    dstc = (dstc0, dstc1)
    bufd = (bufd0, bufd1)
    bufe = (bufe0, bufe1)
    semi = (semi0, semi1)
    semg = (semg0, semg1)
    last = GCH - 1

    def a_stage(j, s):
        eb = ebase0 + jnp.minimum(j, last) * CG
        pltpu.async_copy(src1.at[pl.ds(eb, CG)], srcc[s], semi[s])
        pltpu.async_copy(dst1.at[pl.ds(eb, CG)], dstc[s], semi[s])

    def wait_i(s):
        pltpu.make_async_copy(src1.at[pl.ds(0, CG)], srcc[s], semi[s]).wait()
        pltpu.make_async_copy(dst1.at[pl.ds(0, CG)], dstc[s], semi[s]).wait()

    def b_stage(j, s):
        wait_i(s)
        pltpu.async_copy(dh.at[srcc[s]], bufd[s], semg[s])
        pltpu.async_copy(eh.at[dstc[s]], bufe[s], semg[s])

    def wait_g(s):
        pltpu.make_async_copy(dh.at[srcc[s]], bufd[s], semg[s]).wait()
        pltpu.make_async_copy(eh.at[dstc[s]], bufe[s], semg[s]).wait()

    def c_stage(j, s):
        wait_g(s)
        a_stage(j + 2, s)

        def row(r, _):
            for c in range(D // 16):
                sl = pl.ds(c * 16, 16)
                bufd[s][r, sl] = bufd[s][r, sl] + bufe[s][r, sl]
            return 0

        lax.fori_loop(0, CG, row, 0)
        pltpu.sync_copy(bufd[s], g_out.at[pl.ds(ebase0 + j * CG, CG)])

    a_stage(0, 0)
    b_stage(0, 0)
    a_stage(1, 1)

    def body(k, _):
        j0 = 2 * k
        j1 = j0 + 1
        b_stage(j1, 1)
        c_stage(j0, 0)
        b_stage(j0 + 2, 0)
        c_stage(j1, 1)
        return 0

    lax.fori_loop(0, GCH // 2, body, 0)
    wait_g(0)
    wait_i(1)


def _gather_call(dh, eh, src1, dst1):
    mesh = plsc.VectorSubcoreMesh(core_axis_name="c", subcore_axis_name="s",
                                  num_cores=2, num_subcores=16)
    kern = pl.kernel(
        _gather_body,
        out_type=jax.ShapeDtypeStruct((NE, D), jnp.float32),
        mesh=mesh,
        scratch_types=[
            pltpu.VMEM((CG,), jnp.int32),
            pltpu.VMEM((CG,), jnp.int32),
            pltpu.VMEM((CG,), jnp.int32),
            pltpu.VMEM((CG,), jnp.int32),
            pltpu.VMEM((CG, D), jnp.float32),
            pltpu.VMEM((CG, D), jnp.float32),
            pltpu.VMEM((CG, D), jnp.float32),
            pltpu.VMEM((CG, D), jnp.float32),
            pltpu.SemaphoreType.DMA,
            pltpu.SemaphoreType.DMA,
            pltpu.SemaphoreType.DMA,
            pltpu.SemaphoreType.DMA,
        ],
    )
    return kern(dh, eh, src1, dst1)


def _edge_body(e_ref, g_ref, cw, cb, eij_out, sig_out, stats_out):
    i = pl.program_id(0)
    ce = jnp.dot(e_ref[...], cw[...], preferred_element_type=jnp.float32)
    eij = ce + cb[...] + g_ref[...]
    eij_out[...] = eij
    sig_out[...] = 1.0 / (1.0 + jnp.exp(-eij))

    @pl.when(i == 0)
    def _():
        stats_out[...] = jnp.zeros_like(stats_out)

    s = jnp.sum(eij, axis=0, keepdims=True)
    s2 = jnp.sum(eij * eij, axis=0, keepdims=True)
    stats_out[0:1, :] = stats_out[0:1, :] + s
    stats_out[1:2, :] = stats_out[1:2, :] + s2


def _edge_call(e, g, cw, cb):
    return pl.pallas_call(
        _edge_body,
        grid=(GRID,),
        in_specs=[
            pl.BlockSpec((EB, D), lambda i: (i, 0)),
            pl.BlockSpec((EB, D), lambda i: (i, 0)),
            pl.BlockSpec((D, D), lambda i: (0, 0)),
            pl.BlockSpec((1, D), lambda i: (0, 0)),
        ],
        out_specs=[
            pl.BlockSpec((EB, D), lambda i: (i, 0)),
            pl.BlockSpec((EB, D), lambda i: (i, 0)),
            pl.BlockSpec((8, D), lambda i: (0, 0)),
        ],
        out_shape=(
            jax.ShapeDtypeStruct((NE, D), jnp.float32),
            jax.ShapeDtypeStruct((NE, D), jnp.float32),
            jax.ShapeDtypeStruct((8, D), jnp.float32),
        ),
    )(e, g, cw, cb)


def _zero_acc(sid, zbuf, acc):
    def zrow(r, _):
        for c in range(D // 16):
            zbuf[r, pl.ds(c * 16, 16)] = jnp.zeros((16,), jnp.float32)
        return 0

    lax.fori_loop(0, ZR, zrow, 0)
    for k in range(RPT // ZR):
        pltpu.sync_copy(zbuf, acc.at[pl.ds(sid * RPT + k * ZR, ZR)])


def _remap(cid, dstc, dstl):
    for c5 in range(C // 16):
        sl = pl.ds(c5 * 16, 16)
        d = dstc[sl] - cid * NH
        ok = (d >= 0) & (d < NH)
        dstl[sl] = jnp.where(ok, d, NH)


def _scatter_num_body(sig, bh, src1, dst1, out,
                      srcc0, srcc1, dstc0, dstc1, dstl0, dstl1,
                      sbuf0, sbuf1, bbuf0, bbuf1, zbuf, acc,
                      semi0, semi1, sems0, sems1, semb0, semb1):
    cid = lax.axis_index("c")
    sid = lax.axis_index("s")
    srcc = (srcc0, srcc1)
    dstc = (dstc0, dstc1)
    dstl = (dstl0, dstl1)
    sbuf = (sbuf0, sbuf1)
    bbuf = (bbuf0, bbuf1)
    semi = (semi0, semi1)
    sems = (sems0, sems1)
    semb = (semb0, semb1)
    last = SCH - 1
    _zero_acc(sid, zbuf, acc)
    plsc.subcore_barrier()

    def a_idx(j, s):
        eb = sid * EPT + jnp.minimum(j, last) * C
        pltpu.async_copy(src1.at[pl.ds(eb, C)], srcc[s], semi[s])
        pltpu.async_copy(dst1.at[pl.ds(eb, C)], dstc[s], semi[s])

    def a_sig(j, s):
        eb = sid * EPT + jnp.minimum(j, last) * C
        pltpu.async_copy(sig.at[pl.ds(eb, C)], sbuf[s], sems[s])

    def wait_i(s):
        pltpu.make_async_copy(src1.at[pl.ds(0, C)], srcc[s], semi[s]).wait()
        pltpu.make_async_copy(dst1.at[pl.ds(0, C)], dstc[s], semi[s]).wait()

    def wait_s(s):
        pltpu.make_async_copy(sig.at[pl.ds(0, C)], sbuf[s], sems[s]).wait()

    def wait_b(s):
        pltpu.make_async_copy(bh.at[srcc[s]], bbuf[s], semb[s]).wait()

    def b_stage(j, s):
        wait_i(s)
        pltpu.async_copy(bh.at[srcc[s]], bbuf[s], semb[s])

    def c_stage(j, s):
        wait_s(s)
        wait_b(s)
        _remap(cid, dstc[s], dstl[s])
        a_idx(j + 2, s)

        def row(r, _):
            for c in range(D // 16):
                sl = pl.ds(c * 16, 16)
                bbuf[s][r, sl] = bbuf[s][r, sl] * sbuf[s][r, sl]
            return 0

        lax.fori_loop(0, C, row, 0)
        a_sig(j + 2, s)
        pltpu.sync_copy(bbuf[s], acc.at[dstl[s]], add=True)

    a_idx(0, 0)
    a_sig(0, 0)
    b_stage(0, 0)
    a_idx(1, 1)
    a_sig(1, 1)

    def body(k, _):
        j0 = 2 * k
        j1 = j0 + 1
        b_stage(j1, 1)
        c_stage(j0, 0)
        b_stage(j0 + 2, 0)
        c_stage(j1, 1)
        return 0

    lax.fori_loop(0, SCH // 2, body, 0)
    wait_b(0)
    wait_s(0)
    wait_i(1)
    wait_s(1)
    plsc.subcore_barrier()
    base = cid * NH + sid * RPT
    pltpu.sync_copy(acc.at[pl.ds(sid * RPT, RPT)], out.at[pl.ds(base, RPT)])


def _scatter_den_body(sig, dst1, out,
                      dstc0, dstc1, dstl0, dstl1, sbuf0, sbuf1, zbuf, acc,
                      semi0, semi1, sems0, sems1):
    cid = lax.axis_index("c")
    sid = lax.axis_index("s")
    dstc = (dstc0, dstc1)
    dstl = (dstl0, dstl1)
    sbuf = (sbuf0, sbuf1)
    semi = (semi0, semi1)
    sems = (sems0, sems1)
    last = SCH - 1
    _zero_acc(sid, zbuf, acc)
    plsc.subcore_barrier()

    def a_idx(j, s):
        eb = sid * EPT + jnp.minimum(j, last) * C
        pltpu.async_copy(dst1.at[pl.ds(eb, C)], dstc[s], semi[s])

    def a_sig(j, s):
        eb = sid * EPT + jnp.minimum(j, last) * C
        pltpu.async_copy(sig.at[pl.ds(eb, C)], sbuf[s], sems[s])

    def wait_i(s):
        pltpu.make_async_copy(dst1.at[pl.ds(0, C)], dstc[s], semi[s]).wait()

    def wait_s(s):
        pltpu.make_async_copy(sig.at[pl.ds(0, C)], sbuf[s], sems[s]).wait()

    def c_stage(j, s):
        wait_i(s)
        wait_s(s)
        _remap(cid, dstc[s], dstl[s])
        a_idx(j + 2, s)
        pltpu.sync_copy(sbuf[s], acc.at[dstl[s]], add=True)
        a_sig(j + 2, s)

    a_idx(0, 0)
    a_sig(0, 0)
    a_idx(1, 1)
    a_sig(1, 1)

    def body(k, _):
        c_stage(2 * k, 0)
        c_stage(2 * k + 1, 1)
        return 0

    lax.fori_loop(0, SCH // 2, body, 0)
    wait_i(0)
    wait_s(0)
    wait_i(1)
    wait_s(1)
    plsc.subcore_barrier()
    base = cid * NH + sid * RPT
    pltpu.sync_copy(acc.at[pl.ds(sid * RPT, RPT)], out.at[pl.ds(base, RPT)])


_SC_MESH_KW = dict(core_axis_name="c", subcore_axis_name="s",
                   num_cores=2, num_subcores=16)


def _scatter_num_call(sig, bh, src1, dst1):
    kern = pl.kernel(
        _scatter_num_body,
        out_type=jax.ShapeDtypeStruct((NP, D), jnp.float32),
        mesh=plsc.VectorSubcoreMesh(**_SC_MESH_KW),
        scratch_types=(
            [pltpu.VMEM((C,), jnp.int32)] * 6
            + [pltpu.VMEM((C, D), jnp.float32)] * 4
            + [pltpu.VMEM((ZR, D), jnp.float32),
               pltpu.VMEM_SHARED((NHP, D), jnp.float32)]
            + [pltpu.SemaphoreType.DMA] * 6
        ),
    )
    return kern(sig, bh, src1, dst1)


def _scatter_den_call(sig, dst1):
    kern = pl.kernel(
        _scatter_den_body,
        out_type=jax.ShapeDtypeStruct((NP, D), jnp.float32),
        mesh=plsc.VectorSubcoreMesh(**_SC_MESH_KW),
        scratch_types=(
            [pltpu.VMEM((C,), jnp.int32)] * 4
            + [pltpu.VMEM((C, D), jnp.float32)] * 2
            + [pltpu.VMEM((ZR, D), jnp.float32),
               pltpu.VMEM_SHARED((NHP, D), jnp.float32)]
            + [pltpu.SemaphoreType.DMA] * 4
        ),
    )
    return kern(sig, dst1)


def _eout_body(e_ref, eij_ref, stats, ge, be, out_ref):
    mu = stats[0:1, :] * (1.0 / NE)
    var = stats[1:2, :] * (1.0 / NE) - mu * mu
    scale = ge[...] * lax.rsqrt(var + 1e-5)
    out_ref[...] = e_ref[...] + (eij_ref[...] - mu) * scale + be[...]


def _eout_call(e, eij, stats, ge, be):
    return pl.pallas_call(
        _eout_body,
        grid=(GRID,),
        in_specs=[
            pl.BlockSpec((EB, D), lambda i: (i, 0)),
            pl.BlockSpec((EB, D), lambda i: (i, 0)),
            pl.BlockSpec((8, D), lambda i: (0, 0)),
            pl.BlockSpec((1, D), lambda i: (0, 0)),
            pl.BlockSpec((1, D), lambda i: (0, 0)),
        ],
        out_specs=pl.BlockSpec((EB, D), lambda i: (i, 0)),
        out_shape=jax.ShapeDtypeStruct((NE, D), jnp.float32),
    )(e, eij, stats, ge, be)


def _hout_body(h_ref, ah_ref, num_ref, den_ref, norm_ref, gh, bh, out_ref):
    h_new = (ah_ref[...] + num_ref[...] / (den_ref[...] + 1e-6)) * norm_ref[...]
    mu = jnp.mean(h_new, axis=0, keepdims=True)
    diff = h_new - mu
    var = jnp.mean(diff * diff, axis=0, keepdims=True)
    out_ref[...] = h_ref[...] + gh[...] * diff * lax.rsqrt(var + 1e-5) + bh[...]


def _hout_call(h, ah, num, den, norm, gh, bh):
    return pl.pallas_call(
        _hout_body,
        out_shape=jax.ShapeDtypeStruct((N, D), jnp.float32),
    )(h, ah, num, den, norm, gh, bh)


@jax.jit
def kernel(h, e, norm, edge_index, A_w, A_b, B_w, B_b, C_w, C_b,
           D_w, D_b, E_w, E_b, gamma_h, beta_h, gamma_e, beta_e):
    src = edge_index[0].astype(jnp.int32)
    dst = edge_index[1].astype(jnp.int32)

    ah, bh, dh, eh = _node_call(
        h, norm, A_w, A_b.reshape(1, D), B_w, B_b.reshape(1, D),
        D_w, D_b.reshape(1, D), E_w, E_b.reshape(1, D))

    g = _gather_call(dh, eh, src, dst)

    eij, sig, stats = _edge_call(e, g, C_w, C_b.reshape(1, D))

    num = _scatter_num_call(sig, bh, src, dst)
    den = _scatter_den_call(sig, dst)

    e_out = _eout_call(e, eij, stats, gamma_e.reshape(1, D),
                       beta_e.reshape(1, D))
    h_out = _hout_call(h, ah, num[:N], den[:N],
                       norm, gamma_h.reshape(1, D), beta_h.reshape(1, D))
    return (h_out, e_out)


# speedup vs baseline: 1.0696x; 1.0696x over previous
import functools

import jax
import jax.numpy as jnp
from jax import lax
from jax.experimental import pallas as pl
from jax.experimental.pallas import tpu as pltpu
from jax.experimental.pallas import tpu_sc as plsc

N = 10000
NE = 320000
D = 128
H = 64
C = 80
CG = 80
EB = 1600
GRID = NE // EB
NW = 32
EPW = NE // NW
GCH = EPW // CG
EPT = NE // 16
SCH = EPT // C
NH = 5120
NHP = NH + 8
NP = 2 * NH
RPT = NH // 16
ZR = 8


def _node_body(h_ref, norm_ref, aw, ab, bw, bb, dw, db, ew, eb,
               ah_out, bh_out, dh_out, eh_out):
    hn = h_ref[...] * norm_ref[...]
    f32 = jnp.float32
    ah_out[...] = jnp.dot(hn, aw[...], preferred_element_type=f32) + ab[...]
    bh_out[...] = jnp.dot(hn, bw[...], preferred_element_type=f32) + bb[...]
    dh_out[...] = jnp.dot(hn, dw[...], preferred_element_type=f32) + db[...]
    eh_out[...] = jnp.dot(hn, ew[...], preferred_element_type=f32) + eb[...]


def _node_call(h, norm, aw, ab, bw, bb, dw, db, ew, eb):
    return pl.pallas_call(
        _node_body,
        out_shape=(
            jax.ShapeDtypeStruct((N, D), jnp.float32),
            jax.ShapeDtypeStruct((N, D), jnp.float32),
            jax.ShapeDtypeStruct((N, D), jnp.float32),
            jax.ShapeDtypeStruct((N, D), jnp.float32),
        ),
    )(h, norm, aw, ab, bw, bb, dw, db, ew, eb)


def _gather_body(dh, eh, src1, dst1, g_out,
                 srcc0, srcc1, dstc0, dstc1,
                 bufd0, bufd1, bufe0, bufe1,
                 semi0, semi1, semg0, semg1):
    cid = lax.axis_index("c")
    sid = lax.axis_index("s")
    wid = sid * 2 + cid
    ebase0 = wid * EPW
    srcc = (srcc0, srcc1)
    dstc = (dstc0, dstc1)
    bufd = (bufd0, bufd1)
    bufe = (bufe0, bufe1)
    semi = (semi0, semi1)
    semg = (semg0, semg1)
    last = GCH - 1

    def a_stage(j, s):
        eb = ebase0 + jnp.minimum(j, last) * CG
        pltpu.async_copy(src1.at[pl.ds(eb, CG)], srcc[s], semi[s])
        pltpu.async_copy(dst1.at[pl.ds(eb, CG)], dstc[s], semi[s])

    def wait_i(s):
        pltpu.make_async_copy(src1.at[pl.ds(0, CG)], srcc[s], semi[s]).wait()
        pltpu.make_async_copy(dst1.at[pl.ds(0, CG)], dstc[s], semi[s]).wait()

    def b_stage(j, s):
        wait_i(s)
        pltpu.async_copy(dh.at[srcc[s]], bufd[s], semg[s])
        pltpu.async_copy(eh.at[dstc[s]], bufe[s], semg[s])

    def wait_g(s):
        pltpu.make_async_copy(dh.at[srcc[s]], bufd[s], semg[s]).wait()
        pltpu.make_async_copy(eh.at[dstc[s]], bufe[s], semg[s]).wait()

    def c_stage(j, s):
        wait_g(s)
        a_stage(j + 2, s)

        def row(r, _):
            for c in range(D // 16):
                sl = pl.ds(c * 16, 16)
                bufd[s][r, sl] = bufd[s][r, sl] + bufe[s][r, sl]
            return 0

        lax.fori_loop(0, CG, row, 0)
        pltpu.sync_copy(bufd[s], g_out.at[pl.ds(ebase0 + j * CG, CG)])

    a_stage(0, 0)
    b_stage(0, 0)
    a_stage(1, 1)

    def body(k, _):
        j0 = 2 * k
        j1 = j0 + 1
        b_stage(j1, 1)
        c_stage(j0, 0)
        b_stage(j0 + 2, 0)
        c_stage(j1, 1)
        return 0

    lax.fori_loop(0, GCH // 2, body, 0)
    tail = GCH - 1
    b_stage(tail, 1)
    c_stage(tail, 1)
    wait_g(0)
    wait_i(1)


def _gather_call(dh, eh, src1, dst1):
    mesh = plsc.VectorSubcoreMesh(core_axis_name="c", subcore_axis_name="s",
                                  num_cores=2, num_subcores=16)
    kern = pl.kernel(
        _gather_body,
        out_type=jax.ShapeDtypeStruct((NE, D), jnp.float32),
        mesh=mesh,
        scratch_types=[
            pltpu.VMEM((CG,), jnp.int32),
            pltpu.VMEM((CG,), jnp.int32),
            pltpu.VMEM((CG,), jnp.int32),
            pltpu.VMEM((CG,), jnp.int32),
            pltpu.VMEM((CG, D), jnp.float32),
            pltpu.VMEM((CG, D), jnp.float32),
            pltpu.VMEM((CG, D), jnp.float32),
            pltpu.VMEM((CG, D), jnp.float32),
            pltpu.SemaphoreType.DMA,
            pltpu.SemaphoreType.DMA,
            pltpu.SemaphoreType.DMA,
            pltpu.SemaphoreType.DMA,
        ],
    )
    return kern(dh, eh, src1, dst1)


def _edge_body(e_ref, g_ref, cw, cb, eij_out, sig_out, stats_out):
    i = pl.program_id(0)
    ce = jnp.dot(e_ref[...], cw[...], preferred_element_type=jnp.float32)
    eij = ce + cb[...] + g_ref[...]
    eij_out[...] = eij.astype(jnp.bfloat16)
    sig_out[...] = 1.0 / (1.0 + jnp.exp(-eij))

    @pl.when(i == 0)
    def _():
        stats_out[...] = jnp.zeros_like(stats_out)

    s = jnp.sum(eij, axis=0, keepdims=True)
    s2 = jnp.sum(eij * eij, axis=0, keepdims=True)
    stats_out[0:1, :] = stats_out[0:1, :] + s
    stats_out[1:2, :] = stats_out[1:2, :] + s2


def _edge_call(e, g, cw, cb):
    return pl.pallas_call(
        _edge_body,
        grid=(GRID,),
        in_specs=[
            pl.BlockSpec((EB, D), lambda i: (i, 0)),
            pl.BlockSpec((EB, D), lambda i: (i, 0)),
            pl.BlockSpec((D, D), lambda i: (0, 0)),
            pl.BlockSpec((1, D), lambda i: (0, 0)),
        ],
        out_specs=[
            pl.BlockSpec((EB, D), lambda i: (i, 0)),
            pl.BlockSpec((EB, D), lambda i: (i, 0)),
            pl.BlockSpec((8, D), lambda i: (0, 0)),
        ],
        out_shape=(
            jax.ShapeDtypeStruct((NE, D), jnp.bfloat16),
            jax.ShapeDtypeStruct((NE, D), jnp.float32),
            jax.ShapeDtypeStruct((8, D), jnp.float32),
        ),
    )(e, g, cw, cb)


def _zero_acc(sid, zbuf, acc):
    def zrow(r, _):
        for c in range(D // 16):
            zbuf[r, pl.ds(c * 16, 16)] = jnp.zeros((16,), jnp.float32)
        return 0

    lax.fori_loop(0, ZR, zrow, 0)
    for k in range(RPT // ZR):
        pltpu.sync_copy(zbuf, acc.at[pl.ds(sid * RPT + k * ZR, ZR)])


def _remap(cid, dstc, dstl):
    for c5 in range(C // 16):
        sl = pl.ds(c5 * 16, 16)
        d = dstc[sl] - cid * NH
        ok = (d >= 0) & (d < NH)
        dstl[sl] = jnp.where(ok, d, NH)


def _scatter_num_body(sig, bh, src1, dst1, out,
                      srcc0, srcc1, dstc0, dstc1, dstl0, dstl1,
                      sbuf0, sbuf1, bbuf0, bbuf1, zbuf, acc,
                      semi0, semi1, sems0, sems1, semb0, semb1):
    cid = lax.axis_index("c")
    sid = lax.axis_index("s")
    srcc = (srcc0, srcc1)
    dstc = (dstc0, dstc1)
    dstl = (dstl0, dstl1)
    sbuf = (sbuf0, sbuf1)
    bbuf = (bbuf0, bbuf1)
    semi = (semi0, semi1)
    sems = (sems0, sems1)
    semb = (semb0, semb1)
    last = SCH - 1
    _zero_acc(sid, zbuf, acc)
    plsc.subcore_barrier()

    def a_idx(j, s):
        eb = sid * EPT + jnp.minimum(j, last) * C
        pltpu.async_copy(src1.at[pl.ds(eb, C)], srcc[s], semi[s])
        pltpu.async_copy(dst1.at[pl.ds(eb, C)], dstc[s], semi[s])

    def a_sig(j, s):
        eb = sid * EPT + jnp.minimum(j, last) * C
        pltpu.async_copy(sig.at[pl.ds(eb, C)], sbuf[s], sems[s])

    def wait_i(s):
        pltpu.make_async_copy(src1.at[pl.ds(0, C)], srcc[s], semi[s]).wait()
        pltpu.make_async_copy(dst1.at[pl.ds(0, C)], dstc[s], semi[s]).wait()

    def wait_s(s):
        pltpu.make_async_copy(sig.at[pl.ds(0, C)], sbuf[s], sems[s]).wait()

    def wait_b(s):
        pltpu.make_async_copy(bh.at[srcc[s]], bbuf[s], semb[s]).wait()

    def b_stage(j, s):
        wait_i(s)
        pltpu.async_copy(bh.at[srcc[s]], bbuf[s], semb[s])

    def c_stage(j, s):
        wait_s(s)
        wait_b(s)
        _remap(cid, dstc[s], dstl[s])
        a_idx(j + 2, s)

        def row(r, _):
            for c in range(D // 16):
                sl = pl.ds(c * 16, 16)
                bbuf[s][r, sl] = bbuf[s][r, sl] * sbuf[s][r, sl]
            return 0

        lax.fori_loop(0, C, row, 0)
        a_sig(j + 2, s)
        pltpu.sync_copy(bbuf[s], acc.at[dstl[s]], add=True)

    a_idx(0, 0)
    a_sig(0, 0)
    b_stage(0, 0)
    a_idx(1, 1)
    a_sig(1, 1)

    def body(k, _):
        j0 = 2 * k
        j1 = j0 + 1
        b_stage(j1, 1)
        c_stage(j0, 0)
        b_stage(j0 + 2, 0)
        c_stage(j1, 1)
        return 0

    lax.fori_loop(0, SCH // 2, body, 0)
    wait_b(0)
    wait_s(0)
    wait_i(1)
    wait_s(1)
    plsc.subcore_barrier()
    base = cid * NH + sid * RPT
    pltpu.sync_copy(acc.at[pl.ds(sid * RPT, RPT)], out.at[pl.ds(base, RPT)])


def _scatter_den_body(sig, dst1, out,
                      dstc0, dstc1, dstl0, dstl1, sbuf0, sbuf1, zbuf, acc,
                      semi0, semi1, sems0, sems1):
    cid = lax.axis_index("c")
    sid = lax.axis_index("s")
    dstc = (dstc0, dstc1)
    dstl = (dstl0, dstl1)
    sbuf = (sbuf0, sbuf1)
    semi = (semi0, semi1)
    sems = (sems0, sems1)
    last = SCH - 1
    _zero_acc(sid, zbuf, acc)
    plsc.subcore_barrier()

    def a_idx(j, s):
        eb = sid * EPT + jnp.minimum(j, last) * C
        pltpu.async_copy(dst1.at[pl.ds(eb, C)], dstc[s], semi[s])

    def a_sig(j, s):
        eb = sid * EPT + jnp.minimum(j, last) * C
        pltpu.async_copy(sig.at[pl.ds(eb, C)], sbuf[s], sems[s])

    def wait_i(s):
        pltpu.make_async_copy(dst1.at[pl.ds(0, C)], dstc[s], semi[s]).wait()

    def wait_s(s):
        pltpu.make_async_copy(sig.at[pl.ds(0, C)], sbuf[s], sems[s]).wait()

    def c_stage(j, s):
        wait_i(s)
        wait_s(s)
        _remap(cid, dstc[s], dstl[s])
        a_idx(j + 2, s)
        pltpu.sync_copy(sbuf[s], acc.at[dstl[s]], add=True)
        a_sig(j + 2, s)

    a_idx(0, 0)
    a_sig(0, 0)
    a_idx(1, 1)
    a_sig(1, 1)

    def body(k, _):
        c_stage(2 * k, 0)
        c_stage(2 * k + 1, 1)
        return 0

    lax.fori_loop(0, SCH // 2, body, 0)
    wait_i(0)
    wait_s(0)
    wait_i(1)
    wait_s(1)
    plsc.subcore_barrier()
    base = cid * NH + sid * RPT
    pltpu.sync_copy(acc.at[pl.ds(sid * RPT, RPT)], out.at[pl.ds(base, RPT)])


_SC_MESH_KW = dict(core_axis_name="c", subcore_axis_name="s",
                   num_cores=2, num_subcores=16)


def _scatter_num_call(sig, bh, src1, dst1):
    kern = pl.kernel(
        _scatter_num_body,
        out_type=jax.ShapeDtypeStruct((NP, D), jnp.float32),
        mesh=plsc.VectorSubcoreMesh(**_SC_MESH_KW),
        scratch_types=(
            [pltpu.VMEM((C,), jnp.int32)] * 6
            + [pltpu.VMEM((C, D), jnp.float32)] * 4
            + [pltpu.VMEM((ZR, D), jnp.float32),
               pltpu.VMEM_SHARED((NHP, D), jnp.float32)]
            + [pltpu.SemaphoreType.DMA] * 6
        ),
    )
    return kern(sig, bh, src1, dst1)


def _scatter_den_call(sig, dst1):
    kern = pl.kernel(
        _scatter_den_body,
        out_type=jax.ShapeDtypeStruct((NP, D), jnp.float32),
        mesh=plsc.VectorSubcoreMesh(**_SC_MESH_KW),
        scratch_types=(
            [pltpu.VMEM((C,), jnp.int32)] * 4
            + [pltpu.VMEM((C, D), jnp.float32)] * 2
            + [pltpu.VMEM((ZR, D), jnp.float32),
               pltpu.VMEM_SHARED((NHP, D), jnp.float32)]
            + [pltpu.SemaphoreType.DMA] * 4
        ),
    )
    return kern(sig, dst1)


def _eout_body(e_ref, eij_ref, stats, ge, be, out_ref):
    mu = stats[0:1, :] * (1.0 / NE)
    var = stats[1:2, :] * (1.0 / NE) - mu * mu
    scale = ge[...] * lax.rsqrt(var + 1e-5)
    eij = eij_ref[...].astype(jnp.float32)
    out_ref[...] = e_ref[...] + (eij - mu) * scale + be[...]


def _eout_call(e, eij, stats, ge, be):
    return pl.pallas_call(
        _eout_body,
        grid=(GRID,),
        in_specs=[
            pl.BlockSpec((EB, D), lambda i: (i, 0)),
            pl.BlockSpec((EB, D), lambda i: (i, 0)),
            pl.BlockSpec((8, D), lambda i: (0, 0)),
            pl.BlockSpec((1, D), lambda i: (0, 0)),
            pl.BlockSpec((1, D), lambda i: (0, 0)),
        ],
        out_specs=pl.BlockSpec((EB, D), lambda i: (i, 0)),
        out_shape=jax.ShapeDtypeStruct((NE, D), jnp.float32),
    )(e, eij, stats, ge, be)


def _hout_body(h_ref, ah_ref, num_ref, den_ref, norm_ref, gh, bh, out_ref):
    h_new = (ah_ref[...] + num_ref[...] / (den_ref[...] + 1e-6)) * norm_ref[...]
    mu = jnp.mean(h_new, axis=0, keepdims=True)
    diff = h_new - mu
    var = jnp.mean(diff * diff, axis=0, keepdims=True)
    out_ref[...] = h_ref[...] + gh[...] * diff * lax.rsqrt(var + 1e-5) + bh[...]


def _hout_call(h, ah, num, den, norm, gh, bh):
    return pl.pallas_call(
        _hout_body,
        out_shape=jax.ShapeDtypeStruct((N, D), jnp.float32),
    )(h, ah, num, den, norm, gh, bh)


@jax.jit
def kernel(h, e, norm, edge_index, A_w, A_b, B_w, B_b, C_w, C_b,
           D_w, D_b, E_w, E_b, gamma_h, beta_h, gamma_e, beta_e):
    src = edge_index[0].astype(jnp.int32)
    dst = edge_index[1].astype(jnp.int32)

    ah, bh, dh, eh = _node_call(
        h, norm, A_w, A_b.reshape(1, D), B_w, B_b.reshape(1, D),
        D_w, D_b.reshape(1, D), E_w, E_b.reshape(1, D))

    g = _gather_call(dh, eh, src, dst)

    eij, sig, stats = _edge_call(e, g, C_w, C_b.reshape(1, D))

    num = _scatter_num_call(sig, bh, src, dst)
    den = _scatter_den_call(sig, dst)

    e_out = _eout_call(e, eij, stats, gamma_e.reshape(1, D),
                       beta_e.reshape(1, D))
    h_out = _hout_call(h, ah, num[:N], den[:N],
                       norm, gamma_h.reshape(1, D), beta_h.reshape(1, D))
    return (h_out, e_out)


# speedup vs baseline: 1.0697x; 1.0001x over previous
import jax
import jax.numpy as jnp
from jax import lax
from jax.experimental import pallas as pl
from jax.experimental.pallas import tpu as pltpu
from jax.experimental.pallas import tpu_sc as plsc

N = 10000
NE = 320000
D = 128
C = 80
CG = 80
EB = 1600
GRID = NE // EB
NW = 32
EPW = NE // NW
GCH = EPW // CG
EPT = NE // 16
SCH = EPT // C
NH = 5120
NHP = NH + 8
NP = 2 * NH
RPT = NH // 16
ZR = 8


def _node_body(h_ref, norm_ref, aw, ab, bw, bb, dw, db, ew, eb,
               ah_out, bh_out, dh_out, eh_out):
    hn = h_ref[...] * norm_ref[...]
    f32 = jnp.float32
    ah_out[...] = jnp.dot(hn, aw[...], preferred_element_type=f32) + ab[...]
    bh_out[...] = jnp.dot(hn, bw[...], preferred_element_type=f32) + bb[...]
    dh_out[...] = jnp.dot(hn, dw[...], preferred_element_type=f32) + db[...]
    eh_out[...] = jnp.dot(hn, ew[...], preferred_element_type=f32) + eb[...]


def _node_call(h, norm, aw, ab, bw, bb, dw, db, ew, eb):
    return pl.pallas_call(
        _node_body,
        out_shape=(
            jax.ShapeDtypeStruct((N, D), jnp.float32),
            jax.ShapeDtypeStruct((N, D), jnp.float32),
            jax.ShapeDtypeStruct((N, D), jnp.float32),
            jax.ShapeDtypeStruct((N, D), jnp.float32),
        ),
    )(h, norm, aw, ab, bw, bb, dw, db, ew, eb)


def _gather_body(dh, eh, src1, dst1, g_out,
                 srcc0, srcc1, dstc0, dstc1,
                 bufd0, bufd1, bufe0, bufe1,
                 semi0, semi1, semg0, semg1):
    cid = lax.axis_index("c")
    sid = lax.axis_index("s")
    wid = sid * 2 + cid
    ebase0 = wid * EPW
    srcc = (srcc0, srcc1)
    dstc = (dstc0, dstc1)
    bufd = (bufd0, bufd1)
    bufe = (bufe0, bufe1)
    semi = (semi0, semi1)
    semg = (semg0, semg1)
    last = GCH - 1

    def a_stage(j, s):
        eb = ebase0 + jnp.minimum(j, last) * CG
        pltpu.async_copy(src1.at[pl.ds(eb, CG)], srcc[s], semi[s])
        pltpu.async_copy(dst1.at[pl.ds(eb, CG)], dstc[s], semi[s])

    def wait_i(s):
        pltpu.make_async_copy(src1.at[pl.ds(0, CG)], srcc[s], semi[s]).wait()
        pltpu.make_async_copy(dst1.at[pl.ds(0, CG)], dstc[s], semi[s]).wait()

    def b_stage(j, s):
        wait_i(s)
        pltpu.async_copy(dh.at[srcc[s]], bufd[s], semg[s])
        pltpu.async_copy(eh.at[dstc[s]], bufe[s], semg[s])

    def wait_g(s):
        pltpu.make_async_copy(dh.at[srcc[s]], bufd[s], semg[s]).wait()
        pltpu.make_async_copy(eh.at[dstc[s]], bufe[s], semg[s]).wait()

    def c_stage(j, s):
        wait_g(s)
        a_stage(j + 2, s)

        def row(r, _):
            for c in range(D // 16):
                sl = pl.ds(c * 16, 16)
                bufd[s][r, sl] = bufd[s][r, sl] + bufe[s][r, sl]
            return 0

        lax.fori_loop(0, CG, row, 0)
        pltpu.sync_copy(bufd[s], g_out.at[pl.ds(ebase0 + j * CG, CG)])

    a_stage(0, 0)
    b_stage(0, 0)
    a_stage(1, 1)

    def body(k, _):
        j0 = 2 * k
        j1 = j0 + 1
        b_stage(j1, 1)
        c_stage(j0, 0)
        b_stage(j0 + 2, 0)
        c_stage(j1, 1)
        return 0

    lax.fori_loop(0, GCH // 2, body, 0)
    tail = GCH - 1
    b_stage(tail, 1)
    c_stage(tail, 1)
    wait_g(0)
    wait_i(1)


def _gather_call(dh, eh, src1, dst1):
    mesh = plsc.VectorSubcoreMesh(core_axis_name="c", subcore_axis_name="s",
                                  num_cores=2, num_subcores=16)
    kern = pl.kernel(
        _gather_body,
        out_type=jax.ShapeDtypeStruct((NE, D), jnp.float32),
        mesh=mesh,
        scratch_types=[
            pltpu.VMEM((CG,), jnp.int32),
            pltpu.VMEM((CG,), jnp.int32),
            pltpu.VMEM((CG,), jnp.int32),
            pltpu.VMEM((CG,), jnp.int32),
            pltpu.VMEM((CG, D), jnp.float32),
            pltpu.VMEM((CG, D), jnp.float32),
            pltpu.VMEM((CG, D), jnp.float32),
            pltpu.VMEM((CG, D), jnp.float32),
            pltpu.SemaphoreType.DMA,
            pltpu.SemaphoreType.DMA,
            pltpu.SemaphoreType.DMA,
            pltpu.SemaphoreType.DMA,
        ],
    )
    return kern(dh, eh, src1, dst1)


def _edge_body(e_ref, g_ref, cw, cb, eij_out, sig_out, stats_out):
    i = pl.program_id(0)
    ce = jnp.dot(e_ref[...], cw[...], preferred_element_type=jnp.float32)
    eij = ce + cb[...] + g_ref[...]
    eij_out[...] = eij.astype(jnp.bfloat16)
    sig_out[...] = 1.0 / (1.0 + jnp.exp(-eij))

    @pl.when(i == 0)
    def _():
        stats_out[...] = jnp.zeros_like(stats_out)

    s = jnp.sum(eij, axis=0, keepdims=True)
    s2 = jnp.sum(eij * eij, axis=0, keepdims=True)
    stats_out[0:1, :] = stats_out[0:1, :] + s
    stats_out[1:2, :] = stats_out[1:2, :] + s2


def _edge_call(e, g, cw, cb):
    return pl.pallas_call(
        _edge_body,
        grid=(GRID,),
        in_specs=[
            pl.BlockSpec((EB, D), lambda i: (i, 0)),
            pl.BlockSpec((EB, D), lambda i: (i, 0)),
            pl.BlockSpec((D, D), lambda i: (0, 0)),
            pl.BlockSpec((1, D), lambda i: (0, 0)),
        ],
        out_specs=[
            pl.BlockSpec((EB, D), lambda i: (i, 0)),
            pl.BlockSpec((EB, D), lambda i: (i, 0)),
            pl.BlockSpec((8, D), lambda i: (0, 0)),
        ],
        out_shape=(
            jax.ShapeDtypeStruct((NE, D), jnp.bfloat16),
            jax.ShapeDtypeStruct((NE, D), jnp.float32),
            jax.ShapeDtypeStruct((8, D), jnp.float32),
        ),
    )(e, g, cw, cb)


def _zero_acc(sid, zbuf, acc):
    def zrow(r, _):
        for c in range(D // 16):
            zbuf[r, pl.ds(c * 16, 16)] = jnp.zeros((16,), jnp.float32)
        return 0

    lax.fori_loop(0, ZR, zrow, 0)
    for k in range(RPT // ZR):
        pltpu.sync_copy(zbuf, acc.at[pl.ds(sid * RPT + k * ZR, ZR)])


def _remap(cid, dstc, dstl):
    for c5 in range(C // 16):
        sl = pl.ds(c5 * 16, 16)
        d = dstc[sl] - cid * NH
        ok = (d >= 0) & (d < NH)
        dstl[sl] = jnp.where(ok, d, NH)


def _scatter_num_body(sig, bh, src1, dst1, out,
                      srcc0, srcc1, dstc0, dstc1, dstl0, dstl1,
                      sbuf0, sbuf1, bbuf0, bbuf1, zbuf, acc,
                      semi0, semi1, sems0, sems1, semb0, semb1):
    cid = lax.axis_index("c")
    sid = lax.axis_index("s")
    srcc = (srcc0, srcc1)
    dstc = (dstc0, dstc1)
    dstl = (dstl0, dstl1)
    sbuf = (sbuf0, sbuf1)
    bbuf = (bbuf0, bbuf1)
    semi = (semi0, semi1)
    sems = (sems0, sems1)
    semb = (semb0, semb1)
    last = SCH - 1
    _zero_acc(sid, zbuf, acc)
    plsc.subcore_barrier()

    def a_idx(j, s):
        eb = sid * EPT + jnp.minimum(j, last) * C
        pltpu.async_copy(src1.at[pl.ds(eb, C)], srcc[s], semi[s])
        pltpu.async_copy(dst1.at[pl.ds(eb, C)], dstc[s], semi[s])

    def a_sig(j, s):
        eb = sid * EPT + jnp.minimum(j, last) * C
        pltpu.async_copy(sig.at[pl.ds(eb, C)], sbuf[s], sems[s])

    def wait_i(s):
        pltpu.make_async_copy(src1.at[pl.ds(0, C)], srcc[s], semi[s]).wait()
        pltpu.make_async_copy(dst1.at[pl.ds(0, C)], dstc[s], semi[s]).wait()

    def wait_s(s):
        pltpu.make_async_copy(sig.at[pl.ds(0, C)], sbuf[s], sems[s]).wait()

    def wait_b(s):
        pltpu.make_async_copy(bh.at[srcc[s]], bbuf[s], semb[s]).wait()

    def b_stage(j, s):
        wait_i(s)
        pltpu.async_copy(bh.at[srcc[s]], bbuf[s], semb[s])

    def c_stage(j, s):
        wait_s(s)
        wait_b(s)
        _remap(cid, dstc[s], dstl[s])
        a_idx(j + 2, s)

        def row(r, _):
            for c in range(D // 16):
                sl = pl.ds(c * 16, 16)
                bbuf[s][r, sl] = bbuf[s][r, sl] * sbuf[s][r, sl]
            return 0

        lax.fori_loop(0, C, row, 0)
        a_sig(j + 2, s)
        pltpu.sync_copy(bbuf[s], acc.at[dstl[s]], add=True)

    a_idx(0, 0)
    a_sig(0, 0)
    b_stage(0, 0)
    a_idx(1, 1)
    a_sig(1, 1)

    def body(k, _):
        j0 = 2 * k
        j1 = j0 + 1
        b_stage(j1, 1)
        c_stage(j0, 0)
        b_stage(j0 + 2, 0)
        c_stage(j1, 1)
        return 0

    lax.fori_loop(0, SCH // 2, body, 0)
    wait_b(0)
    wait_s(0)
    wait_i(1)
    wait_s(1)
    plsc.subcore_barrier()
    base = cid * NH + sid * RPT
    pltpu.sync_copy(acc.at[pl.ds(sid * RPT, RPT)], out.at[pl.ds(base, RPT)])


def _scatter_den_body(sig, dst1, out,
                      dstc0, dstc1, dstl0, dstl1, sbuf0, sbuf1, zbuf, acc,
                      semi0, semi1, sems0, sems1):
    cid = lax.axis_index("c")
    sid = lax.axis_index("s")
    dstc = (dstc0, dstc1)
    dstl = (dstl0, dstl1)
    sbuf = (sbuf0, sbuf1)
    semi = (semi0, semi1)
    sems = (sems0, sems1)
    last = SCH - 1
    _zero_acc(sid, zbuf, acc)
    plsc.subcore_barrier()

    def a_idx(j, s):
        eb = sid * EPT + jnp.minimum(j, last) * C
        pltpu.async_copy(dst1.at[pl.ds(eb, C)], dstc[s], semi[s])

    def a_sig(j, s):
        eb = sid * EPT + jnp.minimum(j, last) * C
        pltpu.async_copy(sig.at[pl.ds(eb, C)], sbuf[s], sems[s])

    def wait_i(s):
        pltpu.make_async_copy(dst1.at[pl.ds(0, C)], dstc[s], semi[s]).wait()

    def wait_s(s):
        pltpu.make_async_copy(sig.at[pl.ds(0, C)], sbuf[s], sems[s]).wait()

    def c_stage(j, s):
        wait_i(s)
        wait_s(s)
        _remap(cid, dstc[s], dstl[s])
        a_idx(j + 2, s)
        pltpu.sync_copy(sbuf[s], acc.at[dstl[s]], add=True)
        a_sig(j + 2, s)

    a_idx(0, 0)
    a_sig(0, 0)
    a_idx(1, 1)
    a_sig(1, 1)

    def body(k, _):
        c_stage(2 * k, 0)
        c_stage(2 * k + 1, 1)
        return 0

    lax.fori_loop(0, SCH // 2, body, 0)
    wait_i(0)
    wait_s(0)
    wait_i(1)
    wait_s(1)
    plsc.subcore_barrier()
    base = cid * NH + sid * RPT
    pltpu.sync_copy(acc.at[pl.ds(sid * RPT, RPT)], out.at[pl.ds(base, RPT)])


_SC_MESH_KW = dict(core_axis_name="c", subcore_axis_name="s",
                   num_cores=2, num_subcores=16)


def _scatter_num_call(sig, bh, src1, dst1):
    kern = pl.kernel(
        _scatter_num_body,
        out_type=jax.ShapeDtypeStruct((NP, D), jnp.float32),
        mesh=plsc.VectorSubcoreMesh(**_SC_MESH_KW),
        scratch_types=(
            [pltpu.VMEM((C,), jnp.int32)] * 6
            + [pltpu.VMEM((C, D), jnp.float32)] * 4
            + [pltpu.VMEM((ZR, D), jnp.float32),
               pltpu.VMEM_SHARED((NHP, D), jnp.float32)]
            + [pltpu.SemaphoreType.DMA] * 6
        ),
    )
    return kern(sig, bh, src1, dst1)


def _scatter_den_call(sig, dst1):
    kern = pl.kernel(
        _scatter_den_body,
        out_type=jax.ShapeDtypeStruct((NP, D), jnp.float32),
        mesh=plsc.VectorSubcoreMesh(**_SC_MESH_KW),
        scratch_types=(
            [pltpu.VMEM((C,), jnp.int32)] * 4
            + [pltpu.VMEM((C, D), jnp.float32)] * 2
            + [pltpu.VMEM((ZR, D), jnp.float32),
               pltpu.VMEM_SHARED((NHP, D), jnp.float32)]
            + [pltpu.SemaphoreType.DMA] * 4
        ),
    )
    return kern(sig, dst1)


def _eout_body(e_ref, eij_ref, stats, ge, be, out_ref):
    mu = stats[0:1, :] * (1.0 / NE)
    var = stats[1:2, :] * (1.0 / NE) - mu * mu
    scale = ge[...] * lax.rsqrt(var + 1e-5)
    eij = eij_ref[...].astype(jnp.float32)
    out_ref[...] = e_ref[...] + (eij - mu) * scale + be[...]


def _eout_call(e, eij, stats, ge, be):
    return pl.pallas_call(
        _eout_body,
        grid=(GRID,),
        in_specs=[
            pl.BlockSpec((EB, D), lambda i: (i, 0)),
            pl.BlockSpec((EB, D), lambda i: (i, 0)),
            pl.BlockSpec((8, D), lambda i: (0, 0)),
            pl.BlockSpec((1, D), lambda i: (0, 0)),
            pl.BlockSpec((1, D), lambda i: (0, 0)),
        ],
        out_specs=pl.BlockSpec((EB, D), lambda i: (i, 0)),
        out_shape=jax.ShapeDtypeStruct((NE, D), jnp.float32),
    )(e, eij, stats, ge, be)


def _hout_body(h_ref, ah_ref, num_ref, den_ref, norm_ref, gh, bh, out_ref):
    h_new = (ah_ref[...] + num_ref[...] / (den_ref[...] + 1e-6)) * norm_ref[...]
    mu = jnp.mean(h_new, axis=0, keepdims=True)
    diff = h_new - mu
    var = jnp.mean(diff * diff, axis=0, keepdims=True)
    out_ref[...] = h_ref[...] + gh[...] * diff * lax.rsqrt(var + 1e-5) + bh[...]


def _hout_call(h, ah, num, den, norm, gh, bh):
    return pl.pallas_call(
        _hout_body,
        out_shape=jax.ShapeDtypeStruct((N, D), jnp.float32),
    )(h, ah, num, den, norm, gh, bh)


@jax.jit
def kernel(h, e, norm, edge_index, A_w, A_b, B_w, B_b, C_w, C_b,
           D_w, D_b, E_w, E_b, gamma_h, beta_h, gamma_e, beta_e):
    src = edge_index[0].astype(jnp.int32)
    dst = edge_index[1].astype(jnp.int32)

    ah, bh, dh, eh = _node_call(
        h, norm, A_w, A_b.reshape(1, D), B_w, B_b.reshape(1, D),
        D_w, D_b.reshape(1, D), E_w, E_b.reshape(1, D))

    g = _gather_call(dh, eh, src, dst)

    eij, sig, stats = _edge_call(e, g, C_w, C_b.reshape(1, D))

    num = _scatter_num_call(sig, bh, src, dst)
    den = _scatter_den_call(sig, dst)

    e_out = _eout_call(e, eij, stats, gamma_e.reshape(1, D),
                       beta_e.reshape(1, D))
    h_out = _hout_call(h, ah, num[:N], den[:N],
                       norm, gamma_h.reshape(1, D), beta_h.reshape(1, D))
    return (h_out, e_out)
